# Initial kernel scaffold; baseline (speedup 1.0000x reference)
import jax
import jax.numpy as jnp
from jax.experimental import pallas as pl


def kernel(z1, z2, edge_index):
    raise NotImplementedError("write your pallas kernel here")


# speedup vs baseline: 1.1058x; 1.1058x over previous
import functools

import jax
import jax.numpy as jnp
from jax import lax
from jax.experimental import pallas as pl
from jax.experimental.pallas import tpu as pltpu
from jax.experimental.pallas import tpu_sc as plsc

N_NODES = 10000
N_EDGES = 320000
D_FEAT = 128

NUM_CORES = 2
NUM_SUBCORES = 16
NW = NUM_CORES * NUM_SUBCORES
EPW = N_EDGES // NW
CHUNK = 80
NCHUNK = EPW // CHUNK
L = 16


def _edge_decoder(z1_hbm, z2_hbm, ei_hbm, out_hbm,
                  sidx, didx, arows, brows, outv, sem_a, sem_b):
    c = lax.axis_index("c")
    s = lax.axis_index("s")
    wid = s * NUM_CORES + c
    base_w = wid * EPW

    def chunk_body(ci, carry):
        base = base_w + ci * CHUNK
        pltpu.sync_copy(ei_hbm.at[pl.ds(base, CHUNK)], sidx)
        pltpu.sync_copy(ei_hbm.at[pl.ds(N_EDGES + base, CHUNK)], didx)
        cp_a = pltpu.async_copy(z1_hbm.at[sidx], arows, sem_a)
        cp_b = pltpu.async_copy(z2_hbm.at[didx], brows, sem_b)
        cp_a.wait()
        cp_b.wait()

        iota = lax.iota(jnp.int32, L)
        for g in range(CHUNK // L):
            rows = iota + (g * L)
            cols = jnp.full((L,), 0, jnp.int32)
            va = plsc.load_gather(arows, [rows, cols])
            vb = plsc.load_gather(brows, [rows, cols])
            acc = va * vb
            for _ in range(1, D_FEAT):
                cols = cols + 1
                va = plsc.load_gather(arows, [rows, cols])
                vb = plsc.load_gather(brows, [rows, cols])
                acc = acc + va * vb
            outv[pl.ds(g * L, L)] = 1.0 / (1.0 + jnp.exp(-acc))

        pltpu.sync_copy(outv, out_hbm.at[pl.ds(base, CHUNK)])
        return carry

    lax.fori_loop(0, NCHUNK, chunk_body, 0)


def kernel(z1, z2, edge_index):
    ei = edge_index.astype(jnp.int32).reshape(-1)
    mesh = plsc.VectorSubcoreMesh(core_axis_name="c", subcore_axis_name="s")
    f = functools.partial(
        pl.kernel,
        mesh=mesh,
        compiler_params=pltpu.CompilerParams(needs_layout_passes=False),
        out_type=jax.ShapeDtypeStruct((N_EDGES,), jnp.float32),
        scratch_types=[
            pltpu.VMEM((CHUNK,), jnp.int32),
            pltpu.VMEM((CHUNK,), jnp.int32),
            pltpu.VMEM((CHUNK, D_FEAT), jnp.float32),
            pltpu.VMEM((CHUNK, D_FEAT), jnp.float32),
            pltpu.VMEM((CHUNK,), jnp.float32),
            pltpu.SemaphoreType.DMA,
            pltpu.SemaphoreType.DMA,
        ],
    )(_edge_decoder)
    return f(z1, z2, ei)


# speedup vs baseline: 1.3487x; 1.2197x over previous
import functools

import jax
import jax.numpy as jnp
from jax import lax
from jax.experimental import pallas as pl
from jax.experimental.pallas import tpu as pltpu
from jax.experimental.pallas import tpu_sc as plsc

N_NODES = 10000
N_EDGES = 320000
D_FEAT = 128

NUM_CORES = 2
NUM_SUBCORES = 16
NW = NUM_CORES * NUM_SUBCORES
EPW = N_EDGES // NW
CHUNK = 80
NCHUNK = EPW // CHUNK
NB = 2
L = 16


def _edge_decoder(z1_hbm, z2_hbm, ei_hbm, out_hbm,
                  sall, dall, arows, brows, ov, ga, gb, so):
    c_ax = lax.axis_index("c")
    s_ax = lax.axis_index("s")
    wid = s_ax * NUM_CORES + c_ax
    base_w = wid * EPW

    pltpu.sync_copy(ei_hbm.at[pl.ds(base_w, EPW)], sall)
    pltpu.sync_copy(ei_hbm.at[pl.ds(N_EDGES + base_w, EPW)], dall)

    def issue_gather(cc, b):
        pltpu.async_copy(
            z1_hbm.at[sall.at[pl.ds(cc * CHUNK, CHUNK)]], arows[b], ga[b])
        pltpu.async_copy(
            z2_hbm.at[dall.at[pl.ds(cc * CHUNK, CHUNK)]], brows[b], gb[b])

    def wait_gather(b):
        pltpu.make_async_copy(z1_hbm.at[pl.ds(0, CHUNK)], arows[b],
                              ga[b]).wait()
        pltpu.make_async_copy(z2_hbm.at[pl.ds(0, CHUNK)], brows[b],
                              gb[b]).wait()

    def wait_out(b):
        pltpu.make_async_copy(ov[b], out_hbm.at[pl.ds(base_w, CHUNK)],
                              so[b]).wait()

    UNROLL = 8

    def compute(cc, b):
        def g_body(g, carry):
            rows = lax.iota(jnp.int32, L) + g * L

            def d_body(d, acc):
                for u in range(UNROLL):
                    cols = jnp.full((L,), d * UNROLL + u, jnp.int32)
                    va = plsc.load_gather(arows[b], [rows, cols])
                    vb = plsc.load_gather(brows[b], [rows, cols])
                    acc = acc + va * vb
                return acc

            acc = lax.fori_loop(0, D_FEAT // UNROLL, d_body,
                                jnp.zeros((L,), jnp.float32))
            ov[b][pl.ds(g * L, L)] = 1.0 / (1.0 + jnp.exp(-acc))
            return carry

        lax.fori_loop(0, CHUNK // L, g_body, 0)
        pltpu.async_copy(ov[b],
                         out_hbm.at[pl.ds(base_w + cc * CHUNK, CHUNK)],
                         so[b])

    for b in range(NB):
        issue_gather(jnp.int32(b), b)

    n_main = (NCHUNK - 1) // NB

    def turn(t, carry):
        for b in range(NB):
            cc = t * NB + b
            wait_gather(b)

            @pl.when(t > 0)
            def _():
                wait_out(b)

            compute(cc, b)
            issue_gather(jnp.minimum(cc + NB, NCHUNK - 1), b)
        return carry

    lax.fori_loop(0, n_main, turn, 0)

    wait_gather(0)
    wait_out(0)
    compute(jnp.int32(NCHUNK - 1), 0)
    for b in range(1, NB):
        wait_gather(b)
        wait_out(b)
    wait_out(0)


def kernel(z1, z2, edge_index):
    ei = edge_index.astype(jnp.int32).reshape(-1)
    mesh = plsc.VectorSubcoreMesh(core_axis_name="c", subcore_axis_name="s")
    f = functools.partial(
        pl.kernel,
        mesh=mesh,
        compiler_params=pltpu.CompilerParams(needs_layout_passes=False),
        out_type=jax.ShapeDtypeStruct((N_EDGES,), jnp.float32),
        scratch_types=[
            pltpu.VMEM((EPW,), jnp.int32),
            pltpu.VMEM((EPW,), jnp.int32),
            [pltpu.VMEM((CHUNK, D_FEAT), jnp.float32) for _ in range(NB)],
            [pltpu.VMEM((CHUNK, D_FEAT), jnp.float32) for _ in range(NB)],
            [pltpu.VMEM((CHUNK,), jnp.float32) for _ in range(NB)],
            [pltpu.SemaphoreType.DMA for _ in range(NB)],
            [pltpu.SemaphoreType.DMA for _ in range(NB)],
            [pltpu.SemaphoreType.DMA for _ in range(NB)],
        ],
    )(_edge_decoder)
    return f(z1, z2, ei)
